# Initial kernel scaffold; baseline (speedup 1.0000x reference)
import jax
import jax.numpy as jnp
from jax.experimental import pallas as pl


def kernel(input, mapRecord, target, output, patchIndex, basis):
    raise NotImplementedError("write your pallas kernel here")


# speedup vs baseline: 8.7031x; 8.7031x over previous
import functools

import jax
import jax.numpy as jnp
from jax import lax
from jax.experimental import pallas as pl
from jax.experimental.pallas import tpu as pltpu
from jax.experimental.pallas import tpu_sc as plsc


def _loss1_sums(inp3, target, output):
    B, C, H, W = target.shape

    def body(in_ref, t_ref, o_ref, sabs_ref, sin_ref, acc_abs, acc_in):
        b = pl.program_id(0)

        @pl.when(b == 0)
        def _():
            acc_abs[...] = jnp.zeros_like(acc_abs)
            acc_in[...] = jnp.zeros_like(acc_in)

        inb = in_ref[0]
        acc = acc_abs[...]
        for c in range(C):
            acc = acc + jnp.abs(jnp.where(inb != 0.0, o_ref[0, c], 0.0) - t_ref[0, c])
        acc_abs[...] = acc
        acc_in[...] = acc_in[...] + inb

        @pl.when(b == pl.num_programs(0) - 1)
        def _():
            sabs_ref[0, 0] = jnp.sum(acc_abs[...])
            sin_ref[0, 0] = jnp.sum(acc_in[...])

    return pl.pallas_call(
        body,
        grid=(B,),
        in_specs=[
            pl.BlockSpec((1, H, W), lambda b: (b, 0, 0)),
            pl.BlockSpec((1, C, H, W), lambda b: (b, 0, 0, 0)),
            pl.BlockSpec((1, C, H, W), lambda b: (b, 0, 0, 0)),
        ],
        out_specs=[
            pl.BlockSpec((1, 1), lambda b: (0, 0), memory_space=pltpu.SMEM),
            pl.BlockSpec((1, 1), lambda b: (0, 0), memory_space=pltpu.SMEM),
        ],
        out_shape=[jax.ShapeDtypeStruct((1, 1), jnp.float32)] * 2,
        scratch_shapes=[pltpu.VMEM((H, W), jnp.float32)] * 2,
    )(inp3, target, output)


def _sc_gather_diff(tflat, oflat, idx3d):
    nw_idx, rows_per_w, lanes = idx3d.shape
    try:
        info = plsc.get_sparse_core_info()
        nc, ns = info.num_cores, info.num_subcores
    except Exception:
        nc, ns = 2, 16
    mesh = plsc.VectorSubcoreMesh(
        core_axis_name="c", subcore_axis_name="s", num_cores=nc, num_subcores=ns
    )

    @functools.partial(
        pl.kernel,
        out_type=jax.ShapeDtypeStruct(idx3d.shape, jnp.float32),
        mesh=mesh,
        scratch_types=[
            pltpu.VMEM((rows_per_w, lanes), jnp.int32),
            pltpu.VMEM((rows_per_w, lanes), jnp.float32),
            pltpu.VMEM((rows_per_w, lanes), jnp.float32),
            pltpu.SemaphoreType.DMA,
        ],
    )
    def k(t_hbm, o_hbm, idx_hbm, out_hbm, idx_v, tv, ov, sem):
        wid = lax.axis_index("s") * nc + lax.axis_index("c")
        pltpu.sync_copy(idx_hbm.at[wid], idx_v)
        copies = []
        for j in range(rows_per_w):
            copies.append(pltpu.async_copy(t_hbm.at[idx_v.at[j]], tv.at[j], sem))
            copies.append(pltpu.async_copy(o_hbm.at[idx_v.at[j]], ov.at[j], sem))
        for cp in copies:
            cp.wait()
        for j in range(rows_per_w):
            for i in range(lanes // 16):
                sl = pl.ds(i * 16, 16)
                tv[j, sl] = tv[j, sl] - ov[j, sl]
        pltpu.sync_copy(tv, out_hbm.at[wid])

    return k(tflat, oflat, idx3d)


def _combine(g2, bT, pI_row, sabs, sin, C):
    BC, KL = g2.shape
    D = bT.shape[0]
    S = bT.shape[1]
    K = KL // D

    def body(g_ref, bT_ref, pI_ref, sabs_ref, sin_ref, loss_ref, l1_ref, l2_ref):
        bTm = bT_ref[...]
        pI = pI_ref[...]
        E = jnp.zeros((BC, S), jnp.float32)
        for k in range(K):
            gk = g_ref[:, k * D:(k + 1) * D]
            Mk = jnp.dot(gk, bTm, preferred_element_type=jnp.float32)
            E = E + Mk * (pI == k).astype(jnp.float32)
        s2 = jnp.sum(jnp.abs(E))
        l1 = sabs_ref[0, 0] / (C * sin_ref[0, 0])
        l2 = s2 / (BC * S)
        l1_ref[0, 0] = l1
        l2_ref[0, 0] = l2
        loss_ref[0, 0] = l1 + l2

    return pl.pallas_call(
        body,
        in_specs=[
            pl.BlockSpec(memory_space=pltpu.VMEM),
            pl.BlockSpec(memory_space=pltpu.VMEM),
            pl.BlockSpec(memory_space=pltpu.VMEM),
            pl.BlockSpec(memory_space=pltpu.SMEM),
            pl.BlockSpec(memory_space=pltpu.SMEM),
        ],
        out_specs=[
            pl.BlockSpec(memory_space=pltpu.SMEM),
            pl.BlockSpec(memory_space=pltpu.SMEM),
            pl.BlockSpec(memory_space=pltpu.SMEM),
        ],
        out_shape=[jax.ShapeDtypeStruct((1, 1), jnp.float32)] * 3,
    )(g2, bT, pI_row, sabs, sin)


def kernel(input, mapRecord, target, output, patchIndex, basis):
    B, C, H, W = output.shape
    L = mapRecord.shape[1]
    S = basis.shape[0]

    inp3 = input.reshape(B, H, W)
    sabs, sin = _loss1_sums(inp3, target, output)

    off = mapRecord[:, :, 0] * W + mapRecord[:, :, 1]
    base = jnp.arange(B * C, dtype=jnp.int32).reshape(B, C) * (H * W)
    idx = base[:, :, None] + off[:, None, :]
    idx3d = idx.reshape(32, -1, 128).astype(jnp.int32)

    g = _sc_gather_diff(target.reshape(-1), output.reshape(-1), idx3d)
    g2 = g.reshape(B * C, L)

    bT = basis.reshape(S, -1).T
    pI_row = patchIndex.reshape(1, S)
    loss, l1, l2 = _combine(g2, bT, pI_row, sabs, sin, C)
    return loss.reshape(()), l1.reshape(()), l2.reshape(())


# speedup vs baseline: 14.6328x; 1.6813x over previous
import functools

import jax
import jax.numpy as jnp
from jax import lax
from jax.experimental import pallas as pl
from jax.experimental.pallas import tpu as pltpu
from jax.experimental.pallas import tpu_sc as plsc


def _loss1_sums(inp3, target, output):
    B, C, H, W = target.shape
    NB = 8

    def body(in_ref, t_ref, o_ref, sabs_ref, sin_ref, acc_abs, acc_in):
        b = pl.program_id(0)

        @pl.when(b == 0)
        def _():
            acc_abs[...] = jnp.zeros_like(acc_abs)
            acc_in[...] = jnp.zeros_like(acc_in)

        acc = acc_abs[...]
        acci = acc_in[...]
        for nb in range(NB):
            inb = in_ref[nb]
            for c in range(C):
                acc = acc + jnp.abs(
                    jnp.where(inb != 0.0, o_ref[nb, c], 0.0) - t_ref[nb, c]
                )
            acci = acci + inb
        acc_abs[...] = acc
        acc_in[...] = acci

        @pl.when(b == pl.num_programs(0) - 1)
        def _():
            sabs_ref[0, 0] = jnp.sum(acc_abs[...])
            sin_ref[0, 0] = jnp.sum(acc_in[...])

    return pl.pallas_call(
        body,
        grid=(B // NB,),
        in_specs=[
            pl.BlockSpec((NB, H, W), lambda b: (b, 0, 0)),
            pl.BlockSpec((NB, C, H, W), lambda b: (b, 0, 0, 0)),
            pl.BlockSpec((NB, C, H, W), lambda b: (b, 0, 0, 0)),
        ],
        out_specs=[
            pl.BlockSpec((1, 1), lambda b: (0, 0), memory_space=pltpu.SMEM),
            pl.BlockSpec((1, 1), lambda b: (0, 0), memory_space=pltpu.SMEM),
        ],
        out_shape=[jax.ShapeDtypeStruct((1, 1), jnp.float32)] * 2,
        scratch_shapes=[pltpu.VMEM((H, W), jnp.float32)] * 2,
    )(inp3, target, output)


def _sc_gather_diff(tflat, oflat, idx3d):
    nw_idx, rows_per_w, lanes = idx3d.shape
    try:
        info = plsc.get_sparse_core_info()
        nc, ns = info.num_cores, info.num_subcores
    except Exception:
        nc, ns = 2, 16
    mesh = plsc.VectorSubcoreMesh(
        core_axis_name="c", subcore_axis_name="s", num_cores=nc, num_subcores=ns
    )

    @functools.partial(
        pl.kernel,
        out_type=jax.ShapeDtypeStruct(idx3d.shape, jnp.float32),
        mesh=mesh,
        scratch_types=[
            pltpu.VMEM((rows_per_w, lanes), jnp.int32),
            pltpu.VMEM((rows_per_w, lanes), jnp.float32),
            pltpu.VMEM((rows_per_w, lanes), jnp.float32),
            pltpu.SemaphoreType.DMA,
        ],
    )
    def k(t_hbm, o_hbm, idx_hbm, out_hbm, idx_v, tv, ov, sem):
        wid = lax.axis_index("s") * nc + lax.axis_index("c")
        pltpu.sync_copy(idx_hbm.at[wid], idx_v)
        copies = []
        for j in range(rows_per_w):
            copies.append(pltpu.async_copy(t_hbm.at[idx_v.at[j]], tv.at[j], sem))
            copies.append(pltpu.async_copy(o_hbm.at[idx_v.at[j]], ov.at[j], sem))
        for cp in copies:
            cp.wait()
        for j in range(rows_per_w):
            for i in range(lanes // 16):
                sl = pl.ds(i * 16, 16)
                tv[j, sl] = tv[j, sl] - ov[j, sl]
        pltpu.sync_copy(tv, out_hbm.at[wid])

    return k(tflat, oflat, idx3d)


def _combine(g2, bT, pI_row, sabs, sin, C):
    BC, KL = g2.shape
    D = bT.shape[0]
    S = bT.shape[1]
    K = KL // D

    def body(g_ref, bT_ref, pI_ref, sabs_ref, sin_ref, loss_ref, l1_ref, l2_ref):
        bTm = bT_ref[...]
        pI = pI_ref[...]
        E = jnp.zeros((BC, S), jnp.float32)
        for k in range(K):
            gk = g_ref[:, k * D:(k + 1) * D]
            Mk = jnp.dot(gk, bTm, preferred_element_type=jnp.float32)
            E = E + Mk * (pI == k).astype(jnp.float32)
        s2 = jnp.sum(jnp.abs(E))
        l1 = sabs_ref[0, 0] / (C * sin_ref[0, 0])
        l2 = s2 / (BC * S)
        l1_ref[0, 0] = l1
        l2_ref[0, 0] = l2
        loss_ref[0, 0] = l1 + l2

    return pl.pallas_call(
        body,
        in_specs=[
            pl.BlockSpec(memory_space=pltpu.VMEM),
            pl.BlockSpec(memory_space=pltpu.VMEM),
            pl.BlockSpec(memory_space=pltpu.VMEM),
            pl.BlockSpec(memory_space=pltpu.SMEM),
            pl.BlockSpec(memory_space=pltpu.SMEM),
        ],
        out_specs=[
            pl.BlockSpec(memory_space=pltpu.SMEM),
            pl.BlockSpec(memory_space=pltpu.SMEM),
            pl.BlockSpec(memory_space=pltpu.SMEM),
        ],
        out_shape=[jax.ShapeDtypeStruct((1, 1), jnp.float32)] * 3,
    )(g2, bT, pI_row, sabs, sin)


def kernel(input, mapRecord, target, output, patchIndex, basis):
    B, C, H, W = output.shape
    L = mapRecord.shape[1]
    S = basis.shape[0]

    inp3 = input.reshape(B, H, W)
    sabs, sin = _loss1_sums(inp3, target, output)

    off = mapRecord[:, :, 0] * W + mapRecord[:, :, 1]
    base = jnp.arange(B * C, dtype=jnp.int32).reshape(B, C) * (H * W)
    idx = base[:, :, None] + off[:, None, :]
    idx3d = idx.reshape(32, -1, 128).astype(jnp.int32)

    g = _sc_gather_diff(target.reshape(-1), output.reshape(-1), idx3d)
    g2 = g.reshape(B * C, L)

    bT = basis.reshape(S, -1).T
    pI_row = patchIndex.reshape(1, S)
    loss, l1, l2 = _combine(g2, bT, pI_row, sabs, sin, C)
    return loss.reshape(()), l1.reshape(()), l2.reshape(())


# speedup vs baseline: 15.4878x; 1.0584x over previous
import functools

import jax
import jax.numpy as jnp
from jax import lax
from jax.experimental import pallas as pl
from jax.experimental.pallas import tpu as pltpu
from jax.experimental.pallas import tpu_sc as plsc


def _loss1_sums(inp3, target, output):
    B, C, H, W = target.shape
    NB = 8

    def body(in_ref, t_ref, o_ref, sabs_ref, sin_ref, acc_abs, acc_in):
        b = pl.program_id(0)

        @pl.when(b == 0)
        def _():
            acc_abs[...] = jnp.zeros_like(acc_abs)
            acc_in[...] = jnp.zeros_like(acc_in)

        acc = acc_abs[...]
        acci = acc_in[...]
        for nb in range(NB):
            inb = in_ref[nb]
            for c in range(C):
                acc = acc + jnp.abs(
                    jnp.where(inb != 0.0, o_ref[nb, c], 0.0) - t_ref[nb, c]
                )
            acci = acci + inb
        acc_abs[...] = acc
        acc_in[...] = acci

        @pl.when(b == pl.num_programs(0) - 1)
        def _():
            sabs_ref[0, 0] = jnp.sum(acc_abs[...])
            sin_ref[0, 0] = jnp.sum(acc_in[...])

    return pl.pallas_call(
        body,
        grid=(B // NB,),
        in_specs=[
            pl.BlockSpec((NB, H, W), lambda b: (b, 0, 0)),
            pl.BlockSpec((NB, C, H, W), lambda b: (b, 0, 0, 0)),
            pl.BlockSpec((NB, C, H, W), lambda b: (b, 0, 0, 0)),
        ],
        out_specs=[
            pl.BlockSpec((1, 1), lambda b: (0, 0), memory_space=pltpu.SMEM),
            pl.BlockSpec((1, 1), lambda b: (0, 0), memory_space=pltpu.SMEM),
        ],
        out_shape=[jax.ShapeDtypeStruct((1, 1), jnp.float32)] * 2,
        scratch_shapes=[pltpu.VMEM((H, W), jnp.float32)] * 2,
    )(inp3, target, output)


def _sc_gather_diff(tflat, oflat, idx3d, out_rows, out_cols):
    nwu, idx_rows, lanes = idx3d.shape
    slab = out_rows // nwu
    per_row = out_cols // lanes
    try:
        info = plsc.get_sparse_core_info()
        nc, ns = info.num_cores, info.num_subcores
    except Exception:
        nc, ns = 2, 16
    mesh = plsc.VectorSubcoreMesh(
        core_axis_name="c", subcore_axis_name="s", num_cores=nc, num_subcores=ns
    )

    @functools.partial(
        pl.kernel,
        out_type=jax.ShapeDtypeStruct((out_rows, out_cols), jnp.float32),
        mesh=mesh,
        scratch_types=[
            pltpu.VMEM((idx_rows, lanes), jnp.int32),
            pltpu.VMEM((slab, out_cols), jnp.float32),
            pltpu.VMEM((slab, out_cols), jnp.float32),
            pltpu.SemaphoreType.DMA,
        ],
    )
    def k(t_hbm, o_hbm, idx_hbm, out_hbm, idx_v, tv, ov, sem):
        wid = lax.axis_index("s") * nc + lax.axis_index("c")

        @pl.when(wid < nwu)
        def _():
            pltpu.sync_copy(idx_hbm.at[wid], idx_v)
            copies = []
            for j in range(idx_rows):
                dst = (j // per_row, pl.ds((j % per_row) * lanes, lanes))
                copies.append(pltpu.async_copy(t_hbm.at[idx_v.at[j]], tv.at[dst], sem))
                copies.append(pltpu.async_copy(o_hbm.at[idx_v.at[j]], ov.at[dst], sem))
            for cp in copies:
                cp.wait()
            for j in range(slab):
                for i in range(out_cols // 16):
                    sl = pl.ds(i * 16, 16)
                    tv[j, sl] = tv[j, sl] - ov[j, sl]
            base = pl.multiple_of(wid * slab, 8)
            pltpu.sync_copy(tv, out_hbm.at[pl.ds(base, slab)])

    return k(tflat, oflat, idx3d)


def _combine(g2, bT, pI_row, sabs, sin, C):
    BC, KL = g2.shape
    D = bT.shape[0]
    S = bT.shape[1]
    K = KL // D

    def body(g_ref, bT_ref, pI_ref, sabs_ref, sin_ref, loss_ref, l1_ref, l2_ref):
        bTm = bT_ref[...]
        pI = pI_ref[...]
        E = jnp.zeros((BC, S), jnp.float32)
        for k in range(K):
            gk = g_ref[:, k * D:(k + 1) * D]
            Mk = jnp.dot(gk, bTm, preferred_element_type=jnp.float32)
            E = E + Mk * (pI == k).astype(jnp.float32)
        s2 = jnp.sum(jnp.abs(E))
        l1 = sabs_ref[0, 0] / (C * sin_ref[0, 0])
        l2 = s2 / (BC * S)
        l1_ref[0, 0] = l1
        l2_ref[0, 0] = l2
        loss_ref[0, 0] = l1 + l2

    return pl.pallas_call(
        body,
        in_specs=[
            pl.BlockSpec(memory_space=pltpu.VMEM),
            pl.BlockSpec(memory_space=pltpu.VMEM),
            pl.BlockSpec(memory_space=pltpu.VMEM),
            pl.BlockSpec(memory_space=pltpu.SMEM),
            pl.BlockSpec(memory_space=pltpu.SMEM),
        ],
        out_specs=[
            pl.BlockSpec(memory_space=pltpu.SMEM),
            pl.BlockSpec(memory_space=pltpu.SMEM),
            pl.BlockSpec(memory_space=pltpu.SMEM),
        ],
        out_shape=[jax.ShapeDtypeStruct((1, 1), jnp.float32)] * 3,
    )(g2, bT, pI_row, sabs, sin)


def kernel(input, mapRecord, target, output, patchIndex, basis):
    B, C, H, W = output.shape
    L = mapRecord.shape[1]
    S = basis.shape[0]

    inp3 = input.reshape(B, H, W)
    sabs, sin = _loss1_sums(inp3, target, output)

    off = mapRecord[:, :, 0] * W + mapRecord[:, :, 1]
    base = jnp.arange(B * C, dtype=jnp.int32).reshape(B, C) * (H * W)
    idx = base[:, :, None] + off[:, None, :]
    idx3d = idx.reshape(24, -1, 128).astype(jnp.int32)

    g2 = _sc_gather_diff(target.reshape(-1), output.reshape(-1), idx3d, B * C, L)

    bT = basis.reshape(S, -1).T
    pI_row = patchIndex.reshape(1, S)
    loss, l1, l2 = _combine(g2, bT, pI_row, sabs, sin, C)
    return loss.reshape(()), l1.reshape(()), l2.reshape(())


# speedup vs baseline: 15.6316x; 1.0093x over previous
import functools

import jax
import jax.numpy as jnp
from jax import lax
from jax.experimental import pallas as pl
from jax.experimental.pallas import tpu as pltpu
from jax.experimental.pallas import tpu_sc as plsc


def _loss1_sums(inp3, target, output):
    B, C, H, W = target.shape
    NB = 16

    def body(in_ref, t_ref, o_ref, sabs_ref, sin_ref, acc_abs, acc_in):
        b = pl.program_id(0)

        @pl.when(b == 0)
        def _():
            acc_abs[...] = jnp.zeros_like(acc_abs)
            acc_in[...] = jnp.zeros_like(acc_in)

        acc = acc_abs[...]
        acci = acc_in[...]
        for nb in range(NB):
            inb = in_ref[nb]
            for c in range(C):
                acc = acc + jnp.abs(
                    jnp.where(inb != 0.0, o_ref[nb, c], 0.0) - t_ref[nb, c]
                )
            acci = acci + inb
        acc_abs[...] = acc
        acc_in[...] = acci

        @pl.when(b == pl.num_programs(0) - 1)
        def _():
            sabs_ref[0, 0] = jnp.sum(acc_abs[...])
            sin_ref[0, 0] = jnp.sum(acc_in[...])

    return pl.pallas_call(
        body,
        grid=(B // NB,),
        in_specs=[
            pl.BlockSpec((NB, H, W), lambda b: (b, 0, 0)),
            pl.BlockSpec((NB, C, H, W), lambda b: (b, 0, 0, 0)),
            pl.BlockSpec((NB, C, H, W), lambda b: (b, 0, 0, 0)),
        ],
        out_specs=[
            pl.BlockSpec((1, 1), lambda b: (0, 0), memory_space=pltpu.SMEM),
            pl.BlockSpec((1, 1), lambda b: (0, 0), memory_space=pltpu.SMEM),
        ],
        out_shape=[jax.ShapeDtypeStruct((1, 1), jnp.float32)] * 2,
        scratch_shapes=[pltpu.VMEM((H, W), jnp.float32)] * 2,
    )(inp3, target, output)


def _sc_gather_diff(tflat, oflat, idx3d, out_rows, out_cols):
    nwu, idx_rows, lanes = idx3d.shape
    slab = out_rows // nwu
    per_row = out_cols // lanes
    try:
        info = plsc.get_sparse_core_info()
        nc, ns = info.num_cores, info.num_subcores
    except Exception:
        nc, ns = 2, 16
    mesh = plsc.VectorSubcoreMesh(
        core_axis_name="c", subcore_axis_name="s", num_cores=nc, num_subcores=ns
    )

    @functools.partial(
        pl.kernel,
        out_type=jax.ShapeDtypeStruct((out_rows, out_cols), jnp.float32),
        mesh=mesh,
        scratch_types=[
            pltpu.VMEM((idx_rows, lanes), jnp.int32),
            pltpu.VMEM((slab, out_cols), jnp.float32),
            pltpu.VMEM((slab, out_cols), jnp.float32),
            pltpu.SemaphoreType.DMA,
        ],
    )
    def k(t_hbm, o_hbm, idx_hbm, out_hbm, idx_v, tv, ov, sem):
        wid = lax.axis_index("s") * nc + lax.axis_index("c")

        @pl.when(wid < nwu)
        def _():
            pltpu.sync_copy(idx_hbm.at[wid], idx_v)
            copies = []
            for j in range(idx_rows):
                dst = (j // per_row, pl.ds((j % per_row) * lanes, lanes))
                copies.append(pltpu.async_copy(t_hbm.at[idx_v.at[j]], tv.at[dst], sem))
                copies.append(pltpu.async_copy(o_hbm.at[idx_v.at[j]], ov.at[dst], sem))
            for cp in copies:
                cp.wait()
            for j in range(slab):
                for i in range(out_cols // 16):
                    sl = pl.ds(i * 16, 16)
                    tv[j, sl] = tv[j, sl] - ov[j, sl]
            base = pl.multiple_of(wid * slab, 8)
            pltpu.sync_copy(tv, out_hbm.at[pl.ds(base, slab)])

    return k(tflat, oflat, idx3d)


def _combine(g2, bT, pI_row, sabs, sin, C):
    BC, KL = g2.shape
    D = bT.shape[0]
    S = bT.shape[1]
    K = KL // D

    def body(g_ref, bT_ref, pI_ref, sabs_ref, sin_ref, loss_ref, l1_ref, l2_ref):
        bTm = bT_ref[...]
        pI = pI_ref[...]
        E = jnp.zeros((BC, S), jnp.float32)
        for k in range(K):
            gk = g_ref[:, k * D:(k + 1) * D]
            Mk = jnp.dot(gk, bTm, preferred_element_type=jnp.float32)
            E = E + Mk * (pI == k).astype(jnp.float32)
        s2 = jnp.sum(jnp.abs(E))
        l1 = sabs_ref[0, 0] / (C * sin_ref[0, 0])
        l2 = s2 / (BC * S)
        l1_ref[0, 0] = l1
        l2_ref[0, 0] = l2
        loss_ref[0, 0] = l1 + l2

    return pl.pallas_call(
        body,
        in_specs=[
            pl.BlockSpec(memory_space=pltpu.VMEM),
            pl.BlockSpec(memory_space=pltpu.VMEM),
            pl.BlockSpec(memory_space=pltpu.VMEM),
            pl.BlockSpec(memory_space=pltpu.SMEM),
            pl.BlockSpec(memory_space=pltpu.SMEM),
        ],
        out_specs=[
            pl.BlockSpec(memory_space=pltpu.SMEM),
            pl.BlockSpec(memory_space=pltpu.SMEM),
            pl.BlockSpec(memory_space=pltpu.SMEM),
        ],
        out_shape=[jax.ShapeDtypeStruct((1, 1), jnp.float32)] * 3,
    )(g2, bT, pI_row, sabs, sin)


def kernel(input, mapRecord, target, output, patchIndex, basis):
    B, C, H, W = output.shape
    L = mapRecord.shape[1]
    S = basis.shape[0]

    inp3 = input.reshape(B, H, W)
    sabs, sin = _loss1_sums(inp3, target, output)

    off = mapRecord[:, :, 0] * W + mapRecord[:, :, 1]
    base = jnp.arange(B * C, dtype=jnp.int32).reshape(B, C) * (H * W)
    idx = base[:, :, None] + off[:, None, :]
    idx3d = idx.reshape(24, -1, 128).astype(jnp.int32)

    g2 = _sc_gather_diff(target.reshape(-1), output.reshape(-1), idx3d, B * C, L)

    bT = basis.reshape(S, -1).T
    pI_row = patchIndex.reshape(1, S)
    loss, l1, l2 = _combine(g2, bT, pI_row, sabs, sin, C)
    return loss.reshape(()), l1.reshape(()), l2.reshape(())
